# baseline (speedup 1.0000x reference)
import jax
import jax.numpy as jnp
from jax.experimental import pallas as pl
from jax.experimental.pallas import tpu as pltpu

_HEADS = 12
_HEAD_DIM = 64
_SEQ = 2048
_NHASH = 2
_NBUCK = 64
_TJ = 512


def _mask_kernel(hid_ref, rot_ref, out_ref, o_scratch):
    j = pl.program_id(1)

    @pl.when(j == 0)
    def _compute_onehots():
        v = hid_ref[0]
        r = rot_ref[0]
        rot = jax.lax.dot_general(
            v, r, (((1,), (0,)), ((), ())),
            preferred_element_type=jnp.float32)
        iota = jax.lax.broadcasted_iota(jnp.int32, (_SEQ, _NBUCK), 1)
        for k in range(_NHASH):
            x = rot[:, 32 * k:32 * k + 32]
            full = jnp.concatenate([x, -x], axis=1)
            mx = jnp.max(full, axis=1, keepdims=True)
            bidx = jnp.min(jnp.where(full == mx, iota, _NBUCK),
                           axis=1, keepdims=True)
            o_scratch[:, _NBUCK * k:_NBUCK * (k + 1)] = (
                iota == bidx).astype(jnp.bfloat16)

    o = o_scratch[...]
    oj = o_scratch[pl.ds(j * _TJ, _TJ), :]
    acc = jax.lax.dot_general(
        o, oj, (((1,), (1,)), ((), ())),
        preferred_element_type=jnp.float32)
    out_ref[0] = acc > 0.5


def kernel(hidden_states, rotations):
    hid3d = hidden_states.reshape(_SEQ, _HEADS, _HEAD_DIM).transpose(1, 0, 2)
    rot3d = rotations.reshape(_HEADS, _HEAD_DIM, _NHASH * (_NBUCK // 2))
    out = pl.pallas_call(
        _mask_kernel,
        grid=(_HEADS, _SEQ // _TJ),
        in_specs=[
            pl.BlockSpec((1, _SEQ, _HEAD_DIM), lambda h, j: (h, 0, 0)),
            pl.BlockSpec((1, _HEAD_DIM, _NHASH * 32), lambda h, j: (h, 0, 0)),
        ],
        out_specs=pl.BlockSpec((1, _SEQ, _TJ), lambda h, j: (h, 0, j)),
        out_shape=jax.ShapeDtypeStruct((_HEADS, _SEQ, _SEQ), jnp.bool_),
        scratch_shapes=[pltpu.VMEM((_SEQ, _NHASH * _NBUCK), jnp.bfloat16)],
    )(hid3d, rot3d)
    return out[None]
